# baseline (speedup 1.0000x reference)
import functools

import jax
import jax.numpy as jnp
from jax import lax
from jax.experimental import pallas as pl
from jax.experimental.pallas import tpu as pltpu
from jax.experimental.pallas import tpu_sc as plsc

N = 10000
E = 320000
D = 128
NG = 64
NCLS = 10

NCORES = 2
NSUB = 16
NW = NCORES * NSUB
EPT = E // NW
CH = 32
CPT = 324
EPTP = CH * CPT
PKS = 16384
DUMP = 16
NDCH = N // DUMP
DCPT = -(-NDCH // NSUB)

def _zero_acc(page_v, acc_sh, sid):
    zero16 = jnp.zeros((16,), jnp.float32)

    @pl.loop(0, DUMP)
    def _(i):
        for j in range(D // 16):
            page_v[i, pl.ds(j * 16, 16)] = zero16

    @pl.loop(0, DCPT)
    def _(k):
        j = k * NSUB + sid

        @pl.when(j < NDCH)
        def _():
            pltpu.sync_copy(page_v, acc_sh.at[pl.ds(j * DUMP, DUMP)])

    plsc.subcore_barrier()


def _dump_acc(page_v, acc_sh, out_hbm, cid, sid):
    plsc.subcore_barrier()

    @pl.loop(0, DCPT)
    def _(k):
        j = k * NSUB + sid

        @pl.when(j < NDCH)
        def _():
            pltpu.sync_copy(acc_sh.at[pl.ds(j * DUMP, DUMP)], page_v)
            pltpu.sync_copy(page_v, out_hbm.at[cid, pl.ds(j * DUMP, DUMP)])


def _edge_body(y_hbm, pkw_hbm, out_hbm,
               sb, db, wbuf, pkwb, r0, r1, r2, r3, acc_sh,
               g0, g1, g2, g3, s0, s1, s2, s3, q0, q1, q2, q3):
    cid = lax.axis_index("c")
    sid = lax.axis_index("s")
    wid = sid * NCORES + cid
    rows = (r0, r1, r2, r3)
    gsem = (g0, g1, g2, g3)
    ssem = (s0, s1, s2, s3)
    psem = (q0, q1, q2, q3)

    _zero_acc(r0.at[pl.ds(0, DUMP)], acc_sh, sid)

    def issue_pkw(k, cc):
        pltpu.async_copy(pkw_hbm.at[wid, cc], pkwb.at[k], psem[k])

    def wait_pkw(k, cc):
        pltpu.make_async_copy(pkw_hbm.at[wid, cc], pkwb.at[k], psem[k]).wait()

    def unpack(k):
        for g in range(CH // 16):
            s = pl.ds(g * 16, 16)
            v = pkwb[k, 0, s]
            sb[k, s] = v & (PKS - 1)
            db[k, s] = lax.shift_right_logical(v, 14)
            wbuf[k, s] = plsc.bitcast(pkwb[k, 1, s], jnp.float32)

    def issue_gather(k):
        pltpu.async_copy(y_hbm.at[sb.at[k]], rows[k], gsem[k])

    def wait_gather(k):
        pltpu.make_async_copy(y_hbm.at[sb.at[k]], rows[k], gsem[k]).wait()

    def issue_scatter(k):
        pltpu.async_copy(rows[k], acc_sh.at[db.at[k]], ssem[k], add=True)

    def wait_scatter(k):
        pltpu.make_async_copy(rows[k], acc_sh.at[db.at[k]], ssem[k]).wait()

    def scale(k):
        wait_gather(k)
        k16 = jnp.full((16,), k, jnp.int32)
        rk = rows[k]

        @pl.loop(0, CH)
        def _(r):
            wbc = plsc.load_gather(wbuf, [k16, jnp.full((16,), r, jnp.int32)])
            for j in range(D // 16):
                rk[r, pl.ds(j * 16, 16)] = rk[r, pl.ds(j * 16, 16)] * wbc

    def body(t, k, drain, feed, pfeed):
        c = t * 4 + k
        x2 = (k + 2) % 4
        if drain:
            wait_scatter(x2)
        if feed:
            wait_pkw(x2, c + 2)
            unpack(x2)
            issue_gather(x2)
        scale(k)
        if pfeed:
            issue_pkw(k, c + 4)
        issue_scatter(k)

    for k in range(4):
        issue_pkw(k, k)
    for k in range(2):
        wait_pkw(k, k)
        unpack(k)
        issue_gather(k)
    body(0, 0, drain=False, feed=True, pfeed=True)
    body(0, 1, drain=False, feed=True, pfeed=True)
    body(0, 2, drain=True, feed=True, pfeed=True)
    body(0, 3, drain=True, feed=True, pfeed=True)

    @pl.loop(1, CPT // 4 - 1)
    def _(t):
        for k in range(4):
            body(t, k, drain=True, feed=True, pfeed=True)

    tl = CPT // 4 - 1
    body(tl, 0, drain=True, feed=True, pfeed=False)
    body(tl, 1, drain=True, feed=True, pfeed=False)
    body(tl, 2, drain=True, feed=False, pfeed=False)
    body(tl, 3, drain=True, feed=False, pfeed=False)
    wait_scatter(2)
    wait_scatter(3)

    _dump_acc(r0.at[pl.ds(0, DUMP)], acc_sh, out_hbm, cid, sid)


@functools.lru_cache(maxsize=None)
def _sc_kernels():
    mesh = plsc.VectorSubcoreMesh(core_axis_name="c", subcore_axis_name="s")
    params = pltpu.CompilerParams(needs_layout_passes=False)
    edge = pl.kernel(
        _edge_body,
        out_type=jax.ShapeDtypeStruct((NCORES, N, D), jnp.float32),
        mesh=mesh,
        compiler_params=params,
        scratch_types=[
            pltpu.VMEM((4, CH), jnp.int32),
            pltpu.VMEM((4, CH), jnp.int32),
            pltpu.VMEM((4, CH), jnp.float32),
            pltpu.VMEM((4, 2, CH), jnp.int32),
            pltpu.VMEM((CH, D), jnp.float32),
            pltpu.VMEM((CH, D), jnp.float32),
            pltpu.VMEM((CH, D), jnp.float32),
            pltpu.VMEM((CH, D), jnp.float32),
            pltpu.VMEM_SHARED((N, D), jnp.float32),
        ] + [pltpu.SemaphoreType.DMA] * 12,
    )
    return (edge,)


def _edge_kernel(y, pkw):
    return _sc_kernels()[0](y, pkw)


RB = 1000
GRID = N // RB


def _tc1_body(x_ref, win_ref, bin_ref, w1_ref, dg0_ref, dg1_ref, y_ref, dis_ref):
    deg = dg0_ref[...] + dg1_ref[...] + 1.0
    dis = lax.rsqrt(deg)
    dis_ref[...] = dis
    h = jnp.maximum(jnp.dot(x_ref[...], win_ref[...],
                            preferred_element_type=jnp.float32) + bin_ref[...], 0.0)
    y_ref[...] = dis * jnp.dot(h, w1_ref[...], preferred_element_type=jnp.float32)


def _tc1(x, W_in, b_in, W1, dg0, dg1):
    return pl.pallas_call(
        _tc1_body,
        grid=(GRID,),
        in_specs=[
            pl.BlockSpec((RB, D), lambda i: (i, 0)),
            pl.BlockSpec((D, D), lambda i: (0, 0)),
            pl.BlockSpec((1, D), lambda i: (0, 0)),
            pl.BlockSpec((D, D), lambda i: (0, 0)),
            pl.BlockSpec((RB, 1), lambda i: (i, 0)),
            pl.BlockSpec((RB, 1), lambda i: (i, 0)),
        ],
        out_specs=[
            pl.BlockSpec((RB, D), lambda i: (i, 0)),
            pl.BlockSpec((RB, 1), lambda i: (i, 0)),
        ],
        out_shape=[
            jax.ShapeDtypeStruct((N, D), jnp.float32),
            jax.ShapeDtypeStruct((N, 1), jnp.float32),
        ],
    )(x, W_in, b_in, W1, dg0, dg1)


def _tc2_body(p0_ref, p1_ref, y_ref, dis_ref, b_ref, w_ref, out_ref):
    dis = dis_ref[...]
    h = jnp.maximum(dis * (p0_ref[...] + p1_ref[...] + y_ref[...]) + b_ref[...], 0.0)
    out_ref[...] = dis * jnp.dot(h, w_ref[...], preferred_element_type=jnp.float32)


def _tc2(p0, p1, y, dis, b, W):
    return pl.pallas_call(
        _tc2_body,
        grid=(GRID,),
        in_specs=[
            pl.BlockSpec((RB, D), lambda i: (i, 0)),
            pl.BlockSpec((RB, D), lambda i: (i, 0)),
            pl.BlockSpec((RB, D), lambda i: (i, 0)),
            pl.BlockSpec((RB, 1), lambda i: (i, 0)),
            pl.BlockSpec((1, D), lambda i: (0, 0)),
            pl.BlockSpec((D, D), lambda i: (0, 0)),
        ],
        out_specs=pl.BlockSpec((RB, D), lambda i: (i, 0)),
        out_shape=jax.ShapeDtypeStruct((N, D), jnp.float32),
    )(p0, p1, y, dis, b, W)


def _tc3_body(p0_ref, p1_ref, y_ref, dis_ref, b_ref, batch_ref, wc_ref, bc_ref,
              out_ref, sums_ref, cnts_ref):
    i = pl.program_id(0)

    @pl.when(i == 0)
    def _():
        sums_ref[...] = jnp.zeros_like(sums_ref)
        cnts_ref[...] = jnp.zeros_like(cnts_ref)

    dis = dis_ref[...]
    h = jnp.maximum(dis * (p0_ref[...] + p1_ref[...] + y_ref[...]) + b_ref[...], 0.0)
    b = batch_ref[...]
    iota = lax.broadcasted_iota(jnp.int32, (RB, NG), 1)
    onehot = (iota == b).astype(jnp.float32)
    dn = (((0,), (0,)), ((), ()))
    sums_ref[...] += lax.dot_general(onehot, h, dn,
                                     preferred_element_type=jnp.float32)
    cnts_ref[...] += lax.dot_general(onehot, jnp.ones((RB, 1), jnp.float32), dn,
                                     preferred_element_type=jnp.float32)

    @pl.when(i == GRID - 1)
    def _():
        rep = sums_ref[...] / jnp.maximum(cnts_ref[...], 1.0)
        out_ref[...] = jnp.dot(rep, wc_ref[...],
                               preferred_element_type=jnp.float32) + bc_ref[...]


def _tc3(p0, p1, y, dis, b, batch2, Wc, bc):
    return pl.pallas_call(
        _tc3_body,
        grid=(GRID,),
        in_specs=[
            pl.BlockSpec((RB, D), lambda i: (i, 0)),
            pl.BlockSpec((RB, D), lambda i: (i, 0)),
            pl.BlockSpec((RB, D), lambda i: (i, 0)),
            pl.BlockSpec((RB, 1), lambda i: (i, 0)),
            pl.BlockSpec((1, D), lambda i: (0, 0)),
            pl.BlockSpec((RB, 1), lambda i: (i, 0)),
            pl.BlockSpec((D, NCLS), lambda i: (0, 0)),
            pl.BlockSpec((1, NCLS), lambda i: (0, 0)),
        ],
        out_specs=pl.BlockSpec((NG, NCLS), lambda i: (0, 0)),
        out_shape=jax.ShapeDtypeStruct((NG, NCLS), jnp.float32),
        scratch_shapes=[
            pltpu.VMEM((NG, D), jnp.float32),
            pltpu.VMEM((NG, 1), jnp.float32),
        ],
    )(p0, p1, y, dis, b, batch2, Wc, bc)


def kernel(x, edge_index, edge_weights, batch, W_in, b_in, W1, b1, W2, b2, Wc, bc):
    src = edge_index[0].astype(jnp.int32)
    dst = edge_index[1].astype(jnp.int32)
    pad = ((0, 0), (0, EPTP - EPT))
    pk3 = jnp.pad((dst * PKS + src).reshape(NW, EPT), pad).reshape(NW, CPT, CH)
    wb3 = jnp.pad(
        lax.bitcast_convert_type(edge_weights.astype(jnp.float32), jnp.int32)
        .reshape(NW, EPT), pad).reshape(NW, CPT, CH)
    pkw = jnp.stack([pk3, wb3], axis=2)

    ones_t = jnp.ones((N, D), jnp.float32)
    deg_parts = _edge_kernel(ones_t, pkw)
    dg0 = lax.slice(deg_parts, (0, 0, 0), (1, N, 1)).reshape(N, 1)
    dg1 = lax.slice(deg_parts, (1, 0, 0), (2, N, 1)).reshape(N, 1)

    y1, dis = _tc1(x, W_in, b_in.reshape(1, D), W1, dg0, dg1)

    p1 = _edge_kernel(y1, pkw)
    y2 = _tc2(p1[0], p1[1], y1, dis, b1.reshape(1, D), W2)

    p2 = _edge_kernel(y2, pkw)
    logits = _tc3(p2[0], p2[1], y2, dis, b2.reshape(1, D),
                  batch.astype(jnp.int32).reshape(N, 1), Wc, bc.reshape(1, NCLS))
    return logits
